# Initial kernel scaffold; baseline (speedup 1.0000x reference)
import jax
import jax.numpy as jnp
from jax.experimental import pallas as pl


def kernel(x, pitch_min, pitch_max, pitch_target, src_mask, use_ground_truth, conv1_w, conv1_b, ln1_g, ln1_b, conv2_w, conv2_b, ln2_g, ln2_b, lin_w, lin_b, emb_table):
    raise NotImplementedError("write your pallas kernel here")


# speedup vs baseline: 16.3271x; 16.3271x over previous
import jax
import jax.numpy as jnp
from jax.experimental import pallas as pl
from jax.experimental.pallas import tpu as pltpu

B, T, CIN, CH, K, NBINS = 16, 2048, 512, 256, 5, 256


def _body(ugt_ref, bins_ref, x_ref, pt_ref, mask_ref, w1_ref, b1_ref,
          g1_ref, gb1_ref, w2_ref, b2_ref, g2_ref, gb2_ref, lw_ref, lb_ref,
          emb_ref, xout_ref, pred_ref, et_ref, ep_ref):
    x = x_ref[0]

    z2 = jnp.zeros((2, CIN), jnp.float32)
    xp = jnp.concatenate([z2, x, z2], axis=0)
    cols = jnp.concatenate([xp[k:k + T] for k in range(K)], axis=1)
    m = jnp.dot(cols, w1_ref[...].reshape(K * CIN, CH),
                preferred_element_type=jnp.float32)
    h = jnp.maximum(m + b1_ref[0], 0.0)
    mu = jnp.mean(h, axis=-1, keepdims=True)
    var = jnp.mean((h - mu) ** 2, axis=-1, keepdims=True)
    h = (h - mu) / jnp.sqrt(var + 1e-5) * g1_ref[0] + gb1_ref[0]

    z2b = jnp.zeros((2, CH), jnp.float32)
    hp = jnp.concatenate([z2b, h, z2b], axis=0)
    cols2 = jnp.concatenate([hp[k:k + T] for k in range(K)], axis=1)
    m2 = jnp.dot(cols2, w2_ref[...].reshape(K * CH, CH),
                 preferred_element_type=jnp.float32)
    h2 = jnp.maximum(m2 + b2_ref[0], 0.0)
    mu2 = jnp.mean(h2, axis=-1, keepdims=True)
    var2 = jnp.mean((h2 - mu2) ** 2, axis=-1, keepdims=True)
    h2 = (h2 - mu2) / jnp.sqrt(var2 + 1e-5) * g2_ref[0] + gb2_ref[0]

    pred = jnp.dot(h2, lw_ref[...], preferred_element_type=jnp.float32)
    pred = pred + lb_ref[0]
    pred = jnp.where(mask_ref[0] != 0.0, 0.0, pred)

    bins = bins_ref[...]
    idx_t = jnp.sum((bins < pt_ref[0]).astype(jnp.int32), axis=1,
                    keepdims=True)
    idx_p = jnp.sum((bins < pred).astype(jnp.int32), axis=1, keepdims=True)

    lanes = jax.lax.broadcasted_iota(jnp.int32, (T, NBINS), 1)
    oh_t = (idx_t == lanes).astype(jnp.float32)
    oh_p = (idx_p == lanes).astype(jnp.float32)
    emb = emb_ref[...]
    et = jnp.dot(oh_t, emb, preferred_element_type=jnp.float32,
                 precision=jax.lax.Precision.HIGHEST)
    ep = jnp.dot(oh_p, emb, preferred_element_type=jnp.float32,
                 precision=jax.lax.Precision.HIGHEST)

    ugt = ugt_ref[0, 0]
    xout_ref[0] = x + jnp.where(ugt != 0, et, ep)
    pred_ref[0] = pred
    et_ref[0] = et
    ep_ref[0] = ep


def kernel(x, pitch_min, pitch_max, pitch_target, src_mask, use_ground_truth,
           conv1_w, conv1_b, ln1_g, ln1_b, conv2_w, conv2_b, ln2_g, ln2_b,
           lin_w, lin_b, emb_table):
    bins = jnp.linspace(pitch_min, pitch_max, NBINS - 1)
    bins_p = jnp.concatenate(
        [bins, jnp.full((1,), jnp.inf, jnp.float32)]).reshape(1, NBINS)
    pt_col = pitch_target.reshape(B, T, 1)
    mask_col = src_mask.astype(jnp.float32).reshape(B, T, 1)
    ugt = jnp.asarray(use_ground_truth, jnp.int32).reshape(1, 1)

    grid = (B,)
    full = lambda *s: pl.BlockSpec(s, lambda b: (0,) * len(s))
    perb = lambda *s: pl.BlockSpec((1,) + s, lambda b: (b,) + (0,) * len(s))

    out_shapes = (
        jax.ShapeDtypeStruct((B, T, CIN), jnp.float32),
        jax.ShapeDtypeStruct((B, T, 1), jnp.float32),
        jax.ShapeDtypeStruct((B, T, CIN), jnp.float32),
        jax.ShapeDtypeStruct((B, T, CIN), jnp.float32),
    )
    x_out, pred_col, et, ep = pl.pallas_call(
        _body,
        grid=grid,
        in_specs=[
            pl.BlockSpec(memory_space=pltpu.SMEM),
            full(1, NBINS),
            perb(T, CIN),
            perb(T, 1),
            perb(T, 1),
            full(K, CIN, CH),
            full(1, CH),
            full(1, CH),
            full(1, CH),
            full(K, CH, CH),
            full(1, CH),
            full(1, CH),
            full(1, CH),
            full(CH, 1),
            full(1, 1),
            full(NBINS, CIN),
        ],
        out_specs=(
            perb(T, CIN),
            perb(T, 1),
            perb(T, CIN),
            perb(T, CIN),
        ),
        out_shape=out_shapes,
        compiler_params=pltpu.CompilerParams(
            dimension_semantics=("arbitrary",)),
    )(ugt, bins_p, x, pt_col, mask_col, conv1_w, conv1_b.reshape(1, CH),
      ln1_g.reshape(1, CH), ln1_b.reshape(1, CH), conv2_w,
      conv2_b.reshape(1, CH), ln2_g.reshape(1, CH), ln2_b.reshape(1, CH),
      lin_w, lin_b.reshape(1, 1), emb_table)

    prediction = pred_col.reshape(B, T)
    return (x_out, prediction, et, ep)
